# baseline (speedup 1.0000x reference)
import functools

import jax
import jax.numpy as jnp
from jax import lax
from jax.experimental import pallas as pl
from jax.experimental.pallas import tpu as pltpu
from jax.experimental.pallas import tpu_sc as plsc

_INFO = plsc.get_sparse_core_info()
_NC = _INFO.num_cores
_NS = _INFO.num_subcores
_NW = _NC * _NS

_CH = 128
_NBUF = 2
_L = 16


def _gather_call(n_j, D, n_i):
    mesh = plsc.VectorSubcoreMesh(core_axis_name="c", subcore_axis_name="s")
    D2 = 2 * D

    @functools.partial(
        pl.kernel,
        mesh=mesh,
        out_type=jax.ShapeDtypeStruct((n_j, D, n_i), jnp.float32),
        compiler_params=pltpu.CompilerParams(needs_layout_passes=False),
        scratch_types=[
            pltpu.VMEM((n_j, _CH), jnp.int32),
            pltpu.VMEM((n_j, _CH), jnp.int32),
            pltpu.VMEM((_NBUF, _CH, D2), jnp.float32),
            pltpu.VMEM((_NBUF, D, _CH), jnp.float32),
            pltpu.SemaphoreType.DMA((_NBUF,)),
            pltpu.SemaphoreType.DMA((_NBUF,)),
        ],
    )
    def body(idx_hbm, pairs_hbm, out_hbm, idx_v, pidx_v, pbufs, tbufs,
             gsems, ssems):
        wid = lax.axis_index("s") * _NC + lax.axis_index("c")
        pltpu.sync_copy(idx_hbm.at[:, wid], idx_v)

        def mk_pidx(k, carry):
            for t in range(_CH // _L):
                sl = pl.ds(t * _L, _L)
                pidx_v[k, sl] = lax.shift_right_logical(idx_v[k, sl], 1)
            return carry
        lax.fori_loop(0, n_j, mk_pidx, 0)

        def issue(j, b):
            pltpu.async_copy(pairs_hbm.at[pidx_v.at[j]], pbufs.at[b],
                             gsems.at[b])

        def wait_gather(b):
            pltpu.make_async_copy(pairs_hbm.at[pl.ds(0, _CH)], pbufs.at[b],
                                  gsems.at[b]).wait()

        def wait_store(b):
            pltpu.make_async_copy(out_hbm.at[0], tbufs.at[b],
                                  ssems.at[b]).wait()

        ibase = wid * _CH
        lanes = jax.lax.iota(jnp.int32, _L)

        def chunk_body(j, b):
            wait_gather(b)
            for tb in range(_CH // _L):
                tsl = pl.ds(tb * _L, _L)
                ivec = idx_v[j, tsl]
                h64 = lax.shift_left(jnp.bitwise_and(ivec, jnp.int32(1)), 6)
                rows = jnp.int32(tb * _L) + lanes
                for d in range(D):
                    v = plsc.load_gather(pbufs.at[b],
                                         [rows, h64 + jnp.int32(d)])
                    tbufs[b, d, tsl] = v
            pltpu.async_copy(tbufs.at[b], out_hbm.at[j, :, pl.ds(ibase, _CH)],
                             ssems.at[b])
            issue((j + _NBUF) % n_j, b)

        for b in range(_NBUF):
            issue(b, b)
        for b in range(_NBUF):
            chunk_body(jnp.int32(b), b)

        def group(g, carry):
            for b in range(_NBUF):
                wait_store(b)
                chunk_body(g * _NBUF + b, b)
            return carry
        lax.fori_loop(1, n_j // _NBUF, group, 0)

        for b in range(_NBUF):
            wait_gather(b)
            wait_store(b)

    return body


def kernel(inlets, weight):
    b0, b1 = inlets.shape
    V, D = weight.shape
    n_i, n_j = b0, b1
    idxT = jnp.swapaxes(inlets, 0, 1).reshape(n_j, _NW, _CH).astype(jnp.int32)
    pairs = weight.reshape(V // 2, 2 * D)
    out3 = _gather_call(n_j, D, n_i)(idxT, pairs)
    return jnp.transpose(out3, (2, 0, 1))


# speedup vs baseline: 1.5014x; 1.5014x over previous
import functools

import jax
import jax.numpy as jnp
from jax import lax
from jax.experimental import pallas as pl
from jax.experimental.pallas import tpu as pltpu
from jax.experimental.pallas import tpu_sc as plsc

_INFO = plsc.get_sparse_core_info()
_NC = _INFO.num_cores
_NS = _INFO.num_subcores
_NW = _NC * _NS

_CH = 128
_NBUF = 2
_L = 16


def _gather_call(n_j, D, n_i):
    mesh = plsc.VectorSubcoreMesh(core_axis_name="c", subcore_axis_name="s")
    D2 = 2 * D

    @functools.partial(
        pl.kernel,
        mesh=mesh,
        out_type=jax.ShapeDtypeStruct((n_j, D, n_i), jnp.float32),
        compiler_params=pltpu.CompilerParams(needs_layout_passes=False),
        scratch_types=[
            pltpu.VMEM((n_j, _CH), jnp.int32),
            pltpu.VMEM((n_j, _CH), jnp.int32),
            pltpu.VMEM((_NBUF, _CH, D2), jnp.float32),
            pltpu.VMEM((_NBUF, D, _CH), jnp.float32),
            pltpu.SemaphoreType.DMA((_NBUF,)),
            pltpu.SemaphoreType.DMA((_NBUF,)),
        ],
    )
    def body(idx_hbm, pairs_hbm, out_hbm, idx_v, pidx_v, pbufs, tbufs,
             gsems, ssems):
        wid = lax.axis_index("s") * _NC + lax.axis_index("c")
        pltpu.sync_copy(idx_hbm.at[:, wid], idx_v)

        def mk_pidx(k, carry):
            for t in range(_CH // _L):
                sl = pl.ds(t * _L, _L)
                pidx_v[k, sl] = lax.shift_right_logical(idx_v[k, sl], 1)
            return carry
        lax.fori_loop(0, n_j, mk_pidx, 0)

        def issue(j, b):
            pltpu.async_copy(pairs_hbm.at[pidx_v.at[j]], pbufs.at[b],
                             gsems.at[b])

        def wait_gather(b):
            pltpu.make_async_copy(pairs_hbm.at[pl.ds(0, _CH)], pbufs.at[b],
                                  gsems.at[b]).wait()

        def wait_store(b):
            pltpu.make_async_copy(out_hbm.at[0], tbufs.at[b],
                                  ssems.at[b]).wait()

        ibase = wid * _CH
        lanes = jax.lax.iota(jnp.int32, _L)

        def chunk_body(j, b):
            wait_gather(b)
            for tb in range(_CH // _L):
                tsl = pl.ds(tb * _L, _L)
                ivec = idx_v[j, tsl]
                h64 = lax.shift_left(jnp.bitwise_and(ivec, jnp.int32(1)), 6)
                rows = jnp.int32(tb * _L) + lanes

                @plsc.parallel_loop(0, D, 1, unroll=8)
                def dloop(d, _b=b, _tsl=tsl, _h64=h64, _rows=rows):
                    v = plsc.load_gather(pbufs.at[_b], [_rows, _h64 + d])
                    tbufs[_b, d, _tsl] = v
            pltpu.async_copy(tbufs.at[b], out_hbm.at[j, :, pl.ds(ibase, _CH)],
                             ssems.at[b])
            issue((j + _NBUF) % n_j, b)

        for b in range(_NBUF):
            issue(b, b)
        for b in range(_NBUF):
            chunk_body(jnp.int32(b), b)

        def group(g, carry):
            for b in range(_NBUF):
                wait_store(b)
                chunk_body(g * _NBUF + b, b)
            return carry
        lax.fori_loop(1, n_j // _NBUF, group, 0)

        for b in range(_NBUF):
            wait_gather(b)
            wait_store(b)

    return body


def kernel(inlets, weight):
    b0, b1 = inlets.shape
    V, D = weight.shape
    n_i, n_j = b0, b1
    idxT = jnp.swapaxes(inlets, 0, 1).reshape(n_j, _NW, _CH).astype(jnp.int32)
    pairs = weight.reshape(V // 2, 2 * D)
    out3 = _gather_call(n_j, D, n_i)(idxT, pairs)
    return jnp.transpose(out3, (2, 0, 1))
